# baseline (speedup 1.0000x reference)
import functools

import jax
import jax.numpy as jnp
from jax import lax
from jax.experimental import pallas as pl
from jax.experimental.pallas import tpu as pltpu
from jax.experimental.pallas import tpu_sc as plsc

VOCAB_ROWS = 30000
DIM = 100

NUM_CORES = 2
NUM_SUBCORES = 16
LANES = 16
NUM_WORKERS = NUM_CORES * NUM_SUBCORES

ROW_BLOCK = 2048
UNROLL = 4


def _tw_body(table_ref, w_ref, b_ref, out_ref):
    acc = jax.lax.dot_general(
        w_ref[...], table_ref[...],
        dimension_numbers=(((1,), (1,)), ((), ())),
        preferred_element_type=jnp.float32,
        precision=jax.lax.Precision.HIGHEST,
    )
    out_ref[...] = acc.reshape(ROW_BLOCK) + b_ref[0]


def _precompute_tw(table, W, b):
    return pl.pallas_call(
        _tw_body,
        grid=(pl.cdiv(VOCAB_ROWS, ROW_BLOCK),),
        in_specs=[
            pl.BlockSpec((ROW_BLOCK, DIM), lambda i: (i, 0)),
            pl.BlockSpec((1, DIM), lambda i: (0, 0)),
            pl.BlockSpec(memory_space=pltpu.SMEM),
        ],
        out_specs=pl.BlockSpec((ROW_BLOCK,), lambda i: (i,)),
        out_shape=jax.ShapeDtypeStruct((VOCAB_ROWS,), jnp.float32),
    )(table, W, b)


def _gather_kernel(batch, seq):
    n_total = batch * seq
    n_per_w = n_total // NUM_WORKERS
    rows_per_w = batch // NUM_WORKERS
    mesh = plsc.VectorSubcoreMesh(
        core_axis_name="c", subcore_axis_name="s",
        num_cores=NUM_CORES, num_subcores=NUM_SUBCORES)

    @functools.partial(
        pl.kernel,
        mesh=mesh,
        out_type=jax.ShapeDtypeStruct((batch, seq, 1), jnp.float32),
        scratch_types=[
            pltpu.VMEM((VOCAB_ROWS,), jnp.float32),
            pltpu.VMEM((rows_per_w, seq), jnp.int32),
            pltpu.VMEM((rows_per_w, seq, 1), jnp.float32),
        ],
        compiler_params=pltpu.CompilerParams(
            needs_layout_passes=False, use_tc_tiling_on_sc=False),
    )
    def gather(tw_hbm, x_hbm, out_hbm, tw_v, x_v, out_v):
        wid = lax.axis_index("s") * NUM_CORES + lax.axis_index("c")
        row0 = wid * rows_per_w
        pltpu.sync_copy(tw_hbm, tw_v)
        pltpu.sync_copy(x_hbm.at[pl.ds(row0, rows_per_w)], x_v)
        starts = [*range(0, seq - LANES, LANES), seq - LANES]
        lane = lax.iota(jnp.int32, LANES)
        zeros = jnp.zeros((LANES,), jnp.int32)
        col_vecs = [s + lane for s in starts]

        def body(r, carry):
            r16 = jnp.full((LANES,), r, jnp.int32)
            for c0, c16 in zip(starts, col_vecs):
                idx16 = x_v[r, pl.ds(c0, LANES)]
                vals = plsc.load_gather(tw_v, [idx16])
                plsc.store_scatter(out_v, [r16, c16, zeros], vals)
            return carry

        lax.fori_loop(0, rows_per_w, body, 0)
        pltpu.sync_copy(out_v, out_hbm.at[pl.ds(row0, rows_per_w)])

    return gather


def kernel(x, table, W, b):
    B, L = x.shape
    tw = _precompute_tw(table, W, b)
    return _gather_kernel(B, L)(tw, x)


# speedup vs baseline: 2.9402x; 2.9402x over previous
import functools

import jax
import jax.numpy as jnp
from jax import lax
from jax.experimental import pallas as pl
from jax.experimental.pallas import tpu as pltpu
from jax.experimental.pallas import tpu_sc as plsc

VOCAB_ROWS = 30000
DIM = 100
XPAD = 128

NUM_CORES = 2
NUM_SUBCORES = 16
LANES = 16
NUM_WORKERS = NUM_CORES * NUM_SUBCORES

ROW_BLOCK = 2048


def _tw_body(table_ref, w_ref, b_ref, out_ref):
    acc = jax.lax.dot_general(
        w_ref[...], table_ref[...],
        dimension_numbers=(((1,), (1,)), ((), ())),
        preferred_element_type=jnp.float32,
    )
    out_ref[...] = acc.reshape(ROW_BLOCK) + b_ref[0]


def _precompute_tw(table, W, b):
    return pl.pallas_call(
        _tw_body,
        grid=(pl.cdiv(VOCAB_ROWS, ROW_BLOCK),),
        in_specs=[
            pl.BlockSpec((ROW_BLOCK, DIM), lambda i: (i, 0)),
            pl.BlockSpec((1, DIM), lambda i: (0, 0)),
            pl.BlockSpec(memory_space=pltpu.SMEM),
        ],
        out_specs=pl.BlockSpec((ROW_BLOCK,), lambda i: (i,)),
        out_shape=jax.ShapeDtypeStruct((VOCAB_ROWS,), jnp.float32),
    )(table, W, b)


def _gather_kernel(batch, seq):
    n_total = batch * seq
    n_per_w = n_total // NUM_WORKERS
    rows_per_w = batch // NUM_WORKERS
    mesh = plsc.VectorSubcoreMesh(
        core_axis_name="c", subcore_axis_name="s",
        num_cores=NUM_CORES, num_subcores=NUM_SUBCORES)

    @functools.partial(
        pl.kernel,
        mesh=mesh,
        out_type=jax.ShapeDtypeStruct((n_total,), jnp.float32),
        scratch_types=[
            pltpu.VMEM((VOCAB_ROWS,), jnp.float32),
            pltpu.VMEM((rows_per_w, XPAD), jnp.int32),
            pltpu.VMEM((n_per_w,), jnp.float32),
        ],
        compiler_params=pltpu.CompilerParams(
            needs_layout_passes=False, use_tc_tiling_on_sc=False),
    )
    def gather(tw_hbm, x_hbm, out_hbm, tw_v, x_v, out_v):
        wid = lax.axis_index("s") * NUM_CORES + lax.axis_index("c")
        row0 = wid * rows_per_w
        pltpu.sync_copy(tw_hbm, tw_v)
        pltpu.sync_copy(x_hbm.at[pl.ds(row0, rows_per_w)], x_v)

        starts = [*range(0, seq - LANES, LANES), seq - LANES]

        def body(r, carry):
            o = r * seq
            for c0 in starts:
                idx16 = x_v[r, pl.ds(c0, LANES)]
                out_v[pl.ds(o + c0, LANES)] = plsc.load_gather(tw_v, [idx16])
            return carry

        lax.fori_loop(0, rows_per_w, body, 0)
        pltpu.sync_copy(out_v, out_hbm.at[pl.ds(wid * n_per_w, n_per_w)])

    return gather


def kernel(x, table, W, b):
    B, L = x.shape
    tw = _precompute_tw(table, W, b)
    x_pad = jnp.pad(x, ((0, 0), (0, XPAD - L)))
    flat = _gather_kernel(B, L)(tw, x_pad)
    return flat.reshape(B, L, 1)


# speedup vs baseline: 3.3853x; 1.1514x over previous
import functools

import jax
import jax.numpy as jnp
from jax import lax
from jax.experimental import pallas as pl
from jax.experimental.pallas import tpu as pltpu
from jax.experimental.pallas import tpu_sc as plsc

VOCAB_ROWS = 30000
DIM = 100
XPAD = 128

NUM_CORES = 2
NUM_SUBCORES = 16
LANES = 16
NUM_WORKERS = NUM_CORES * NUM_SUBCORES

ROW_BLOCK = 6144


def _tw_body(table_ref, w_ref, b_ref, out_ref):
    acc = jax.lax.dot_general(
        w_ref[...], table_ref[...],
        dimension_numbers=(((1,), (1,)), ((), ())),
        preferred_element_type=jnp.float32,
    )
    out_ref[...] = acc.reshape(ROW_BLOCK) + b_ref[0]


def _precompute_tw(table, W, b):
    return pl.pallas_call(
        _tw_body,
        grid=(pl.cdiv(VOCAB_ROWS, ROW_BLOCK),),
        in_specs=[
            pl.BlockSpec((ROW_BLOCK, DIM), lambda i: (i, 0)),
            pl.BlockSpec((1, DIM), lambda i: (0, 0)),
            pl.BlockSpec(memory_space=pltpu.SMEM),
        ],
        out_specs=pl.BlockSpec((ROW_BLOCK,), lambda i: (i,)),
        out_shape=jax.ShapeDtypeStruct((VOCAB_ROWS,), jnp.float32),
    )(table, W, b)


def _gather_kernel(batch, seq):
    n_total = batch * seq
    n_per_w = n_total // NUM_WORKERS
    rows_per_w = batch // NUM_WORKERS
    mesh = plsc.VectorSubcoreMesh(
        core_axis_name="c", subcore_axis_name="s",
        num_cores=NUM_CORES, num_subcores=NUM_SUBCORES)

    @functools.partial(
        pl.kernel,
        mesh=mesh,
        out_type=jax.ShapeDtypeStruct((n_total,), jnp.float32),
        scratch_types=[
            pltpu.VMEM((VOCAB_ROWS,), jnp.float32),
            pltpu.VMEM((rows_per_w * XPAD,), jnp.int32),
            pltpu.VMEM((n_per_w,), jnp.float32),
        ],
        compiler_params=pltpu.CompilerParams(
            needs_layout_passes=False, use_tc_tiling_on_sc=False),
    )
    def gather(tw_hbm, x_hbm, out_hbm, tw_v, x_v, out_v):
        wid = lax.axis_index("s") * NUM_CORES + lax.axis_index("c")
        pltpu.sync_copy(tw_hbm, tw_v)
        pltpu.sync_copy(
            x_hbm.at[pl.ds(wid * rows_per_w * XPAD, rows_per_w * XPAD)], x_v)

        starts = [*range(0, seq - LANES, LANES), seq - LANES]

        @plsc.parallel_loop(0, rows_per_w, 1, unroll=2)
        def body(r):
            for c0 in starts:
                idx16 = x_v[pl.ds(r * XPAD + c0, LANES)]
                out_v[pl.ds(r * seq + c0, LANES)] = plsc.load_gather(
                    tw_v, [idx16])

        pltpu.sync_copy(out_v, out_hbm.at[pl.ds(wid * n_per_w, n_per_w)])

    return gather


def kernel(x, table, W, b):
    B, L = x.shape
    tw = _precompute_tw(table, W, b)
    x_pad = jnp.pad(x, ((0, 0), (0, XPAD - L))).reshape(B * XPAD)
    flat = _gather_kernel(B, L)(tw, x_pad)
    return flat.reshape(B, L, 1)
